# baseline (speedup 1.0000x reference)
import jax
import jax.numpy as jnp
from jax import lax
from jax.experimental import pallas as pl
from jax.experimental.pallas import tpu as pltpu
from jax.experimental.pallas import tpu_sc as plsc

N = 10000
E = 160000
D = 256
H = 128
NC = 2
NS = 16
CH = 128

EPW = E // (NC * NS)
EPT = E // NS
NFULL = N // CH
NTAIL = N - NFULL * CH

def _mesh():
    return plsc.VectorSubcoreMesh(core_axis_name="c", subcore_axis_name="s",
                                  num_cores=NC, num_subcores=NS)


def _zero_vmem(ref, nelem):
    z = jnp.zeros((16,), jnp.float32)

    @pl.loop(0, nelem // 16)
    def _(i):
        ref[pl.ds(i * 16, 16)] = z


EMB_CH = 80
EMB_NCH = N // EMB_CH
HIST_NCH = EPW // CH


def _sc_prep_body(x_hbm, dst_hbm, emb_hbm, h0_hbm, degp_hbm,
                  xv0, xv1, rows0, rows1, dv0, dv1, ones_v, dtail, ones_t,
                  zdeg, acc, es0, es1, hs0, hs1):
    c = lax.axis_index("c")
    s = lax.axis_index("s")
    w = s * NC + c
    xv = (xv0, xv1)
    rows = (rows0, rows1)
    dv = (dv0, dv1)
    esem = (es0, es1)
    hsem = (hs0, hs1)
    ebase = w * EPW

    _zero_vmem(zdeg, CH)
    for k in range(NFULL // NS + 1):
        i = s + k * NS

        @pl.when(i < NFULL)
        def _():
            pltpu.sync_copy(zdeg, acc.at[pl.ds(i * CH, CH)])

    @pl.when(s == NS - 1)
    def _():
        pltpu.sync_copy(zdeg.at[pl.ds(0, NTAIL)], acc.at[pl.ds(NFULL * CH, NTAIL)])

    for r in range(CH // 16):
        ones_v[pl.ds(r * 16, 16)] = jnp.ones((16,), jnp.float32)
    ones_t[...] = jnp.ones((8,), jnp.float32)

    for b in range(2):
        pltpu.async_copy(dst_hbm.at[pl.ds(ebase + b * CH, CH)], dv[b], hsem[b])

    nk = EMB_NCH // (NC * NS) + 1

    for k in range(nk + 1):
        if k < nk:
            ci = w + k * NC * NS
            b = k % 2

            @pl.when(ci < EMB_NCH)
            def _():
                pltpu.sync_copy(x_hbm.at[pl.ds(ci * EMB_CH, EMB_CH)], xv[b])
                pltpu.async_copy(emb_hbm.at[xv[b]], rows[b], esem[b])
        if k >= 1:
            pci = w + (k - 1) * NC * NS
            pb = (k - 1) % 2

            @pl.when(pci < EMB_NCH)
            def _():
                pltpu.make_async_copy(emb_hbm.at[xv[pb]], rows[pb],
                                      esem[pb]).wait()
                pltpu.sync_copy(rows[pb], h0_hbm.at[pl.ds(pci * EMB_CH, EMB_CH)])

    plsc.subcore_barrier()

    @pl.loop(0, HIST_NCH // 2)
    def _(i):
        for b in range(2):
            j = 2 * i + b
            pltpu.make_async_copy(dst_hbm.at[pl.ds(ebase, CH)], dv[b],
                                  hsem[b]).wait()
            pltpu.sync_copy(ones_v, acc.at[dv[b]], add=True)
            j2 = j + 2

            @pl.when(j2 < HIST_NCH)
            def _():
                pltpu.async_copy(dst_hbm.at[pl.ds(ebase + j2 * CH, CH)],
                                 dv[b], hsem[b])

    if HIST_NCH % 2 == 1:
        pltpu.make_async_copy(dst_hbm.at[pl.ds(ebase, CH)], dv[0], hsem[0]).wait()
        pltpu.sync_copy(ones_v, acc.at[dv[0]], add=True)

    rem = EPW - HIST_NCH * CH
    pltpu.sync_copy(dst_hbm.at[pl.ds(ebase + EPW - rem, rem)], dtail)
    pltpu.sync_copy(ones_t, acc.at[dtail], add=True)

    plsc.subcore_barrier()

    @pl.when(s == 0)
    def _():
        pltpu.sync_copy(acc, degp_hbm.at[c])


def _sc_prep(x, dst, emb):
    return pl.kernel(
        _sc_prep_body,
        out_type=(jax.ShapeDtypeStruct((N, D), jnp.float32),
                  jax.ShapeDtypeStruct((NC, N), jnp.float32)),
        mesh=_mesh(),
        scratch_types=[
            pltpu.VMEM((EMB_CH,), jnp.int32),
            pltpu.VMEM((EMB_CH,), jnp.int32),
            pltpu.VMEM((EMB_CH, D), jnp.float32),
            pltpu.VMEM((EMB_CH, D), jnp.float32),
            pltpu.VMEM((CH,), jnp.int32),
            pltpu.VMEM((CH,), jnp.int32),
            pltpu.VMEM((CH,), jnp.float32),
            pltpu.VMEM((8,), jnp.int32),
            pltpu.VMEM((8,), jnp.float32),
            pltpu.VMEM((CH,), jnp.float32),
            pltpu.VMEM_SHARED((N,), jnp.float32),
            pltpu.SemaphoreType.DMA,
            pltpu.SemaphoreType.DMA,
            pltpu.SemaphoreType.DMA,
            pltpu.SemaphoreType.DMA,
        ],
    )(x, dst, emb)


ROWS_PT = (E // CH) // NS
EPT_FULL = ROWS_PT * CH
EXTRA_OFF = NS * EPT_FULL
N_EXTRA = (E - NS * EPT_FULL) // CH
NB = 3


def _sc_agg_body(g_hbm, src_hbm, dst_hbm, zeros_hbm, s_hbm,
                 s0, s1, s2, v0, v1, v2, d0, d1, d2, r0, r1, r2, acc,
                 gs0, gs1, gs2, ds0, ds1, ds2, xs0, xs1, xs2):
    c = lax.axis_index("c")
    s = lax.axis_index("s")
    cN = c * N
    sv = (s0, s1, s2)
    svo = (v0, v1, v2)
    dv = (d0, d1, d2)
    rows = (r0, r1, r2)
    gsem = (gs0, gs1, gs2)
    dsem = (ds0, ds1, ds2)
    xsem = (xs0, xs1, xs2)

    def zero_chunks(do):
        for k in range(NFULL // NS + 1):
            i = s + k * NS

            @pl.when(i < NFULL)
            def _():
                do(zeros_hbm.at[pl.ds(i * CH, CH)], acc.at[pl.ds(i * CH, CH)])

        @pl.when(s == NS - 1)
        def _():
            do(zeros_hbm.at[pl.ds(NFULL * CH, NTAIL)],
               acc.at[pl.ds(NFULL * CH, NTAIL)])

    zero_chunks(lambda a, b: pltpu.async_copy(a, b, gs0))
    zero_chunks(lambda a, b: pltpu.make_async_copy(a, b, gs0).wait())

    ebase = s * EPT_FULL
    nch = jnp.where(s < N_EXTRA, ROWS_PT + 1, ROWS_PT)

    def doff(j):
        return jnp.where(j < ROWS_PT, ebase + j * CH, EXTRA_OFF + s * CH)

    def add_off(b):
        for q in range(CH // 16):
            svo[b][pl.ds(q * 16, 16)] = sv[b][pl.ds(q * 16, 16)] + cN

    for b in range(NB):
        pltpu.async_copy(dst_hbm.at[pl.ds(doff(b), CH)], dv[b], dsem[b])
        pltpu.async_copy(src_hbm.at[pl.ds(doff(b), CH)], sv[b], xsem[b])
    for b in range(NB):
        pltpu.make_async_copy(src_hbm.at[pl.ds(doff(b), CH)], sv[b],
                              xsem[b]).wait()
        add_off(b)
        pltpu.async_copy(g_hbm.at[svo[b]], rows[b], gsem[b])
    for b in range(NB):
        pltpu.async_copy(src_hbm.at[pl.ds(doff(b + NB), CH)], sv[b], xsem[b])

    plsc.subcore_barrier()

    @pl.loop(0, ROWS_PT // NB)
    def _(i):
        for b in range(NB):
            j = i * NB + b
            jn = j + NB
            pltpu.make_async_copy(dst_hbm.at[pl.ds(doff(j), CH)], dv[b],
                                  dsem[b]).wait()
            pltpu.make_async_copy(g_hbm.at[svo[b]], rows[b], gsem[b]).wait()
            pltpu.sync_copy(rows[b], acc.at[dv[b]], add=True)

            @pl.when(jn < nch)
            def _():
                pltpu.make_async_copy(src_hbm.at[pl.ds(doff(jn), CH)], sv[b],
                                      xsem[b]).wait()
                add_off(b)
                pltpu.async_copy(g_hbm.at[svo[b]], rows[b], gsem[b])
                pltpu.async_copy(dst_hbm.at[pl.ds(doff(jn), CH)], dv[b], dsem[b])
                jnn = jn + NB

                @pl.when(jnn < nch)
                def _():
                    pltpu.async_copy(src_hbm.at[pl.ds(doff(jnn), CH)], sv[b],
                                     xsem[b])

    for b in range(ROWS_PT % NB + 1):
        j = (ROWS_PT // NB) * NB + b

        @pl.when(j < nch)
        def _():
            pltpu.make_async_copy(dst_hbm.at[pl.ds(doff(j), CH)], dv[b],
                                  dsem[b]).wait()
            pltpu.make_async_copy(g_hbm.at[svo[b]], rows[b], gsem[b]).wait()
            pltpu.sync_copy(rows[b], acc.at[dv[b]], add=True)

    plsc.subcore_barrier()

    def out_chunks(do):
        for k in range(NFULL // NS + 1):
            i = s + k * NS

            @pl.when(i < NFULL)
            def _():
                do(acc.at[pl.ds(i * CH, CH)], s_hbm.at[c, pl.ds(i * CH, CH)])

        @pl.when(s == NS - 1)
        def _():
            do(acc.at[pl.ds(NFULL * CH, NTAIL)],
               s_hbm.at[c, pl.ds(NFULL * CH, NTAIL)])

    out_chunks(lambda a, b: pltpu.async_copy(a, b, gs0))
    out_chunks(lambda a, b: pltpu.make_async_copy(a, b, gs0).wait())


def _sc_agg(g2, src, dst, zeros):
    return pl.kernel(
        _sc_agg_body,
        out_type=jax.ShapeDtypeStruct((NC, N, H), jnp.float32),
        mesh=_mesh(),
        scratch_types=(
            [pltpu.VMEM((CH,), jnp.int32) for _ in range(NB)]
            + [pltpu.VMEM((CH,), jnp.int32) for _ in range(NB)]
            + [pltpu.VMEM((CH,), jnp.int32) for _ in range(NB)]
            + [pltpu.VMEM((CH, H), jnp.float32) for _ in range(NB)]
            + [pltpu.VMEM_SHARED((N, H), jnp.float32)]
            + [pltpu.SemaphoreType.DMA for _ in range(3 * NB)]
        ),
    )(g2, src, dst, zeros)


BLK = 1000


def _tc0_body(degp_ref, h0_ref, w_ref, g_ref, dinv_ref):
    deg = degp_ref[0] + degp_ref[1] + 1.0
    dinv = lax.rsqrt(deg)
    dinv_ref[...] = dinv
    u = h0_ref[...] * dinv
    g = jnp.dot(u, w_ref[...], preferred_element_type=jnp.float32)
    g_ref[0] = g[:, :H]
    g_ref[1] = g[:, H:]


def _tc0(degp, h0, W1):
    return pl.pallas_call(
        _tc0_body,
        grid=(N // BLK,),
        in_specs=[
            pl.BlockSpec((NC, BLK, 1), lambda i: (0, i, 0)),
            pl.BlockSpec((BLK, D), lambda i: (i, 0)),
            pl.BlockSpec((D, D), lambda i: (0, 0)),
        ],
        out_specs=[
            pl.BlockSpec((NC, BLK, H), lambda i: (0, i, 0)),
            pl.BlockSpec((BLK, 1), lambda i: (i, 0)),
        ],
        out_shape=[
            jax.ShapeDtypeStruct((NC, N, H), jnp.float32),
            jax.ShapeDtypeStruct((N, 1), jnp.float32),
        ],
    )(degp, h0, W1)


def _tc_mid_body(s_ref, g_ref, dinv_ref, b_ref, w_ref, o_ref):
    dinv = dinv_ref[...]
    t = jnp.concatenate([s_ref[0] + g_ref[0], s_ref[1] + g_ref[1]], axis=1)
    h = jnp.maximum(t * dinv + b_ref[...], 0.0)
    g = jnp.dot(h * dinv, w_ref[...], preferred_element_type=jnp.float32)
    o_ref[0] = g[:, :H]
    o_ref[1] = g[:, H:]


def _tc_mid(s, g, dinv, b, W):
    return pl.pallas_call(
        _tc_mid_body,
        grid=(N // BLK,),
        in_specs=[
            pl.BlockSpec((NC, BLK, H), lambda i: (0, i, 0)),
            pl.BlockSpec((NC, BLK, H), lambda i: (0, i, 0)),
            pl.BlockSpec((BLK, 1), lambda i: (i, 0)),
            pl.BlockSpec((1, D), lambda i: (0, 0)),
            pl.BlockSpec((D, D), lambda i: (0, 0)),
        ],
        out_specs=pl.BlockSpec((NC, BLK, H), lambda i: (0, i, 0)),
        out_shape=jax.ShapeDtypeStruct((NC, N, H), jnp.float32),
    )(s, g, dinv, b, W)


def _tc_fin_body(s_ref, g_ref, dinv_ref, b_ref, o_ref):
    t = jnp.concatenate([s_ref[0] + g_ref[0], s_ref[1] + g_ref[1]], axis=1)
    o_ref[...] = t * dinv_ref[...] + b_ref[...]


def _tc_fin(s, g, dinv, b):
    return pl.pallas_call(
        _tc_fin_body,
        grid=(N // BLK,),
        in_specs=[
            pl.BlockSpec((NC, BLK, H), lambda i: (0, i, 0)),
            pl.BlockSpec((NC, BLK, H), lambda i: (0, i, 0)),
            pl.BlockSpec((BLK, 1), lambda i: (i, 0)),
            pl.BlockSpec((1, D), lambda i: (0, 0)),
        ],
        out_specs=pl.BlockSpec((BLK, D), lambda i: (i, 0)),
        out_shape=jax.ShapeDtypeStruct((N, D), jnp.float32),
    )(s, g, dinv, b)


def kernel(x, edge_index, emb_table, W1, b1, W2, b2, W3, b3):
    x = x.astype(jnp.int32)
    src = edge_index[0].astype(jnp.int32)
    dst = edge_index[1].astype(jnp.int32)
    zeros = jnp.zeros((N, H), jnp.float32)

    h0, degp = _sc_prep(x, dst, emb_table)
    degp3 = degp.reshape(NC, N, 1)

    g1, dinv = _tc0(degp3, h0, W1)
    s1 = _sc_agg(g1.reshape(NC * N, H), src, dst, zeros)

    g2 = _tc_mid(s1, g1, dinv, b1.reshape(1, D), W2)
    s2 = _sc_agg(g2.reshape(NC * N, H), src, dst, zeros)

    g3 = _tc_mid(s2, g2, dinv, b2.reshape(1, D), W3)
    s3 = _sc_agg(g3.reshape(NC * N, H), src, dst, zeros)

    return _tc_fin(s3, g3, dinv, b3.reshape(1, D))


# speedup vs baseline: 1.0069x; 1.0069x over previous
import jax
import jax.numpy as jnp
from jax import lax
from jax.experimental import pallas as pl
from jax.experimental.pallas import tpu as pltpu
from jax.experimental.pallas import tpu_sc as plsc

N = 10000
E = 160000
D = 256
H = 128
NC = 2
NS = 16
CH = 128

EPW = E // (NC * NS)
EPT = E // NS
NFULL = N // CH
NTAIL = N - NFULL * CH

def _mesh():
    return plsc.VectorSubcoreMesh(core_axis_name="c", subcore_axis_name="s",
                                  num_cores=NC, num_subcores=NS)


def _zero_vmem(ref, nelem):
    z = jnp.zeros((16,), jnp.float32)

    @pl.loop(0, nelem // 16)
    def _(i):
        ref[pl.ds(i * 16, 16)] = z


EMB_CH = 80
EMB_NCH = N // EMB_CH
HIST_NCH = EPW // CH


def _sc_prep_body(x_hbm, dst_hbm, emb_hbm, h0_hbm, degp_hbm,
                  xv0, xv1, rows0, rows1, dv0, dv1, ones_v, dtail, ones_t,
                  zdeg, acc, es0, es1, hs0, hs1):
    c = lax.axis_index("c")
    s = lax.axis_index("s")
    w = s * NC + c
    xv = (xv0, xv1)
    rows = (rows0, rows1)
    dv = (dv0, dv1)
    esem = (es0, es1)
    hsem = (hs0, hs1)
    ebase = w * EPW

    _zero_vmem(zdeg, CH)
    for k in range(NFULL // NS + 1):
        i = s + k * NS

        @pl.when(i < NFULL)
        def _():
            pltpu.sync_copy(zdeg, acc.at[pl.ds(i * CH, CH)])

    @pl.when(s == NS - 1)
    def _():
        pltpu.sync_copy(zdeg.at[pl.ds(0, NTAIL)], acc.at[pl.ds(NFULL * CH, NTAIL)])

    for r in range(CH // 16):
        ones_v[pl.ds(r * 16, 16)] = jnp.ones((16,), jnp.float32)
    ones_t[...] = jnp.ones((8,), jnp.float32)

    for b in range(2):
        pltpu.async_copy(dst_hbm.at[pl.ds(ebase + b * CH, CH)], dv[b], hsem[b])

    nk = EMB_NCH // (NC * NS) + 1

    for k in range(nk + 1):
        if k < nk:
            ci = w + k * NC * NS
            b = k % 2

            @pl.when(ci < EMB_NCH)
            def _():
                pltpu.sync_copy(x_hbm.at[pl.ds(ci * EMB_CH, EMB_CH)], xv[b])
                pltpu.async_copy(emb_hbm.at[xv[b]], rows[b], esem[b])
        if k >= 1:
            pci = w + (k - 1) * NC * NS
            pb = (k - 1) % 2

            @pl.when(pci < EMB_NCH)
            def _():
                pltpu.make_async_copy(emb_hbm.at[xv[pb]], rows[pb],
                                      esem[pb]).wait()
                pltpu.sync_copy(rows[pb], h0_hbm.at[pl.ds(pci * EMB_CH, EMB_CH)])

    plsc.subcore_barrier()

    @pl.loop(0, HIST_NCH // 2)
    def _(i):
        for b in range(2):
            j = 2 * i + b
            pltpu.make_async_copy(dst_hbm.at[pl.ds(ebase, CH)], dv[b],
                                  hsem[b]).wait()
            pltpu.sync_copy(ones_v, acc.at[dv[b]], add=True)
            j2 = j + 2

            @pl.when(j2 < HIST_NCH)
            def _():
                pltpu.async_copy(dst_hbm.at[pl.ds(ebase + j2 * CH, CH)],
                                 dv[b], hsem[b])

    if HIST_NCH % 2 == 1:
        pltpu.make_async_copy(dst_hbm.at[pl.ds(ebase, CH)], dv[0], hsem[0]).wait()
        pltpu.sync_copy(ones_v, acc.at[dv[0]], add=True)

    rem = EPW - HIST_NCH * CH
    pltpu.sync_copy(dst_hbm.at[pl.ds(ebase + EPW - rem, rem)], dtail)
    pltpu.sync_copy(ones_t, acc.at[dtail], add=True)

    plsc.subcore_barrier()

    @pl.when(s == 0)
    def _():
        pltpu.sync_copy(acc, degp_hbm.at[c])


def _sc_prep(x, dst, emb):
    return pl.kernel(
        _sc_prep_body,
        out_type=(jax.ShapeDtypeStruct((N, D), jnp.float32),
                  jax.ShapeDtypeStruct((NC, N), jnp.float32)),
        mesh=_mesh(),
        scratch_types=[
            pltpu.VMEM((EMB_CH,), jnp.int32),
            pltpu.VMEM((EMB_CH,), jnp.int32),
            pltpu.VMEM((EMB_CH, D), jnp.float32),
            pltpu.VMEM((EMB_CH, D), jnp.float32),
            pltpu.VMEM((CH,), jnp.int32),
            pltpu.VMEM((CH,), jnp.int32),
            pltpu.VMEM((CH,), jnp.float32),
            pltpu.VMEM((8,), jnp.int32),
            pltpu.VMEM((8,), jnp.float32),
            pltpu.VMEM((CH,), jnp.float32),
            pltpu.VMEM_SHARED((N,), jnp.float32),
            pltpu.SemaphoreType.DMA,
            pltpu.SemaphoreType.DMA,
            pltpu.SemaphoreType.DMA,
            pltpu.SemaphoreType.DMA,
        ],
    )(x, dst, emb)


ROWS_PT = (E // CH) // NS
EPT_FULL = ROWS_PT * CH
EXTRA_OFF = NS * EPT_FULL
N_EXTRA = (E - NS * EPT_FULL) // CH
NB = 3


def _sc_agg_body(g_hbm, src_hbm, dst_hbm, zeros_hbm, s_hbm,
                 s0, s1, s2, v0, v1, v2, d0, d1, d2, r0, r1, r2, acc,
                 gs0, gs1, gs2, ds0, ds1, ds2, xs0, xs1, xs2):
    c = lax.axis_index("c")
    s = lax.axis_index("s")
    cN = c * N
    sv = (s0, s1, s2)
    svo = (v0, v1, v2)
    dv = (d0, d1, d2)
    rows = (r0, r1, r2)
    gsem = (gs0, gs1, gs2)
    dsem = (ds0, ds1, ds2)
    xsem = (xs0, xs1, xs2)

    def zero_chunks(do):
        for k in range(NFULL // NS + 1):
            i = s + k * NS

            @pl.when(i < NFULL)
            def _():
                do(zeros_hbm.at[pl.ds(i * CH, CH)], acc.at[pl.ds(i * CH, CH)])

        @pl.when(s == NS - 1)
        def _():
            do(zeros_hbm.at[pl.ds(NFULL * CH, NTAIL)],
               acc.at[pl.ds(NFULL * CH, NTAIL)])

    zero_chunks(lambda a, b: pltpu.async_copy(a, b, gs0))
    zero_chunks(lambda a, b: pltpu.make_async_copy(a, b, gs0).wait())

    ebase = s * EPT_FULL
    nch = jnp.where(s < N_EXTRA, ROWS_PT + 1, ROWS_PT)

    def doff(j):
        return jnp.where(j < ROWS_PT, ebase + j * CH, EXTRA_OFF + s * CH)

    def add_off(b):
        for q in range(CH // 16):
            svo[b][pl.ds(q * 16, 16)] = sv[b][pl.ds(q * 16, 16)] + cN

    plsc.subcore_barrier()

    for b in range(NB):
        pltpu.async_copy(dst_hbm.at[pl.ds(doff(b), CH)], dv[b], dsem[b])
        pltpu.async_copy(src_hbm.at[pl.ds(doff(b), CH)], sv[b], xsem[b])
    for b in range(NB):
        pltpu.make_async_copy(src_hbm.at[pl.ds(doff(b), CH)], sv[b],
                              xsem[b]).wait()
        add_off(b)
        pltpu.async_copy(g_hbm.at[svo[b]], rows[b], gsem[b])
    for b in range(NB):
        pltpu.async_copy(src_hbm.at[pl.ds(doff(b + NB), CH)], sv[b], xsem[b])

    @pl.loop(0, ROWS_PT // NB)
    def _(i):
        for b in range(NB):
            j = i * NB + b
            jn = j + NB
            pltpu.make_async_copy(dst_hbm.at[pl.ds(doff(j), CH)], dv[b],
                                  dsem[b]).wait()
            pltpu.make_async_copy(g_hbm.at[svo[b]], rows[b], gsem[b]).wait()
            pltpu.sync_copy(rows[b], acc.at[dv[b]], add=True)

            @pl.when(jn < nch)
            def _():
                pltpu.make_async_copy(src_hbm.at[pl.ds(doff(jn), CH)], sv[b],
                                      xsem[b]).wait()
                add_off(b)
                pltpu.async_copy(g_hbm.at[svo[b]], rows[b], gsem[b])
                pltpu.async_copy(dst_hbm.at[pl.ds(doff(jn), CH)], dv[b], dsem[b])
                jnn = jn + NB

                @pl.when(jnn < nch)
                def _():
                    pltpu.async_copy(src_hbm.at[pl.ds(doff(jnn), CH)], sv[b],
                                     xsem[b])

    for b in range(ROWS_PT % NB + 1):
        j = (ROWS_PT // NB) * NB + b

        @pl.when(j < nch)
        def _():
            pltpu.make_async_copy(dst_hbm.at[pl.ds(doff(j), CH)], dv[b],
                                  dsem[b]).wait()
            pltpu.make_async_copy(g_hbm.at[svo[b]], rows[b], gsem[b]).wait()
            pltpu.sync_copy(rows[b], acc.at[dv[b]], add=True)

    plsc.subcore_barrier()

    def out_chunks(do):
        for k in range(NFULL // NS + 1):
            i = s + k * NS

            @pl.when(i < NFULL)
            def _():
                do(acc.at[pl.ds(i * CH, CH)], s_hbm.at[c, pl.ds(i * CH, CH)])

        @pl.when(s == NS - 1)
        def _():
            do(acc.at[pl.ds(NFULL * CH, NTAIL)],
               s_hbm.at[c, pl.ds(NFULL * CH, NTAIL)])

    out_chunks(lambda a, b: pltpu.async_copy(a, b, gs0))
    out_chunks(lambda a, b: pltpu.make_async_copy(a, b, gs0).wait())


def _sc_agg(g2, src, dst, zeros):
    return pl.kernel(
        _sc_agg_body,
        out_type=jax.ShapeDtypeStruct((NC, N, H), jnp.float32),
        mesh=_mesh(),
        scratch_types=(
            [pltpu.VMEM((CH,), jnp.int32) for _ in range(NB)]
            + [pltpu.VMEM((CH,), jnp.int32) for _ in range(NB)]
            + [pltpu.VMEM((CH,), jnp.int32) for _ in range(NB)]
            + [pltpu.VMEM((CH, H), jnp.float32) for _ in range(NB)]
            + [pltpu.VMEM_SHARED((N, H), jnp.float32)]
            + [pltpu.SemaphoreType.DMA for _ in range(3 * NB)]
        ),
    )(g2, src, dst, zeros)


BLK = 1000


def _tc0_body(degp_ref, h0_ref, w_ref, g_ref, dinv_ref):
    deg = degp_ref[0] + degp_ref[1] + 1.0
    dinv = lax.rsqrt(deg)
    dinv_ref[...] = dinv
    u = h0_ref[...] * dinv
    g = jnp.dot(u, w_ref[...], preferred_element_type=jnp.float32)
    g_ref[0] = g[:, :H]
    g_ref[1] = g[:, H:]


def _tc0(degp, h0, W1):
    return pl.pallas_call(
        _tc0_body,
        grid=(N // BLK,),
        in_specs=[
            pl.BlockSpec((NC, BLK, 1), lambda i: (0, i, 0)),
            pl.BlockSpec((BLK, D), lambda i: (i, 0)),
            pl.BlockSpec((D, D), lambda i: (0, 0)),
        ],
        out_specs=[
            pl.BlockSpec((NC, BLK, H), lambda i: (0, i, 0)),
            pl.BlockSpec((BLK, 1), lambda i: (i, 0)),
        ],
        out_shape=[
            jax.ShapeDtypeStruct((NC, N, H), jnp.float32),
            jax.ShapeDtypeStruct((N, 1), jnp.float32),
        ],
    )(degp, h0, W1)


def _tc_mid_body(s_ref, g_ref, dinv_ref, b_ref, w_ref, o_ref):
    dinv = dinv_ref[...]
    t = jnp.concatenate([s_ref[0] + g_ref[0], s_ref[1] + g_ref[1]], axis=1)
    h = jnp.maximum(t * dinv + b_ref[...], 0.0)
    g = jnp.dot(h * dinv, w_ref[...], preferred_element_type=jnp.float32)
    o_ref[0] = g[:, :H]
    o_ref[1] = g[:, H:]


def _tc_mid(s, g, dinv, b, W):
    return pl.pallas_call(
        _tc_mid_body,
        grid=(N // BLK,),
        in_specs=[
            pl.BlockSpec((NC, BLK, H), lambda i: (0, i, 0)),
            pl.BlockSpec((NC, BLK, H), lambda i: (0, i, 0)),
            pl.BlockSpec((BLK, 1), lambda i: (i, 0)),
            pl.BlockSpec((1, D), lambda i: (0, 0)),
            pl.BlockSpec((D, D), lambda i: (0, 0)),
        ],
        out_specs=pl.BlockSpec((NC, BLK, H), lambda i: (0, i, 0)),
        out_shape=jax.ShapeDtypeStruct((NC, N, H), jnp.float32),
    )(s, g, dinv, b, W)


def _tc_fin_body(s_ref, g_ref, dinv_ref, b_ref, o_ref):
    t = jnp.concatenate([s_ref[0] + g_ref[0], s_ref[1] + g_ref[1]], axis=1)
    o_ref[...] = t * dinv_ref[...] + b_ref[...]


def _tc_fin(s, g, dinv, b):
    return pl.pallas_call(
        _tc_fin_body,
        grid=(N // BLK,),
        in_specs=[
            pl.BlockSpec((NC, BLK, H), lambda i: (0, i, 0)),
            pl.BlockSpec((NC, BLK, H), lambda i: (0, i, 0)),
            pl.BlockSpec((BLK, 1), lambda i: (i, 0)),
            pl.BlockSpec((1, D), lambda i: (0, 0)),
        ],
        out_specs=pl.BlockSpec((BLK, D), lambda i: (i, 0)),
        out_shape=jax.ShapeDtypeStruct((N, D), jnp.float32),
    )(s, g, dinv, b)


def kernel(x, edge_index, emb_table, W1, b1, W2, b2, W3, b3):
    x = x.astype(jnp.int32)
    src = edge_index[0].astype(jnp.int32)
    dst = edge_index[1].astype(jnp.int32)
    zeros = jnp.zeros((N, H), jnp.float32)

    h0, degp = _sc_prep(x, dst, emb_table)
    degp3 = degp.reshape(NC, N, 1)

    g1, dinv = _tc0(degp3, h0, W1)
    s1 = _sc_agg(g1.reshape(NC * N, H), src, dst, zeros)

    g2 = _tc_mid(s1, g1, dinv, b1.reshape(1, D), W2)
    s2 = _sc_agg(g2.reshape(NC * N, H), src, dst, zeros)

    g3 = _tc_mid(s2, g2, dinv, b2.reshape(1, D), W3)
    s3 = _sc_agg(g3.reshape(NC * N, H), src, dst, zeros)

    return _tc_fin(s3, g3, dinv, b3.reshape(1, D))


# speedup vs baseline: 1.0245x; 1.0175x over previous
import jax
import jax.numpy as jnp
from jax import lax
from jax.experimental import pallas as pl
from jax.experimental.pallas import tpu as pltpu
from jax.experimental.pallas import tpu_sc as plsc

N = 10000
E = 160000
D = 256
H = 128
NC = 2
NS = 16
CH = 128

EPW = E // (NC * NS)
EPT = E // NS
NFULL = N // CH
NTAIL = N - NFULL * CH

def _mesh():
    return plsc.VectorSubcoreMesh(core_axis_name="c", subcore_axis_name="s",
                                  num_cores=NC, num_subcores=NS)


def _zero_vmem(ref, nelem):
    z = jnp.zeros((16,), jnp.float32)

    @pl.loop(0, nelem // 16)
    def _(i):
        ref[pl.ds(i * 16, 16)] = z


EMB_CH = 80
EMB_NCH = N // EMB_CH
HIST_NCH = EPW // CH


def _sc_prep_body(x_hbm, dst_hbm, emb_hbm, h0_hbm, degp_hbm,
                  xv0, xv1, rows0, rows1, dv0, dv1, ones_v, dtail, ones_t,
                  zdeg, acc, es0, es1, hs0, hs1):
    c = lax.axis_index("c")
    s = lax.axis_index("s")
    w = s * NC + c
    xv = (xv0, xv1)
    rows = (rows0, rows1)
    dv = (dv0, dv1)
    esem = (es0, es1)
    hsem = (hs0, hs1)
    ebase = w * EPW

    _zero_vmem(zdeg, CH)
    for k in range(NFULL // NS + 1):
        i = s + k * NS

        @pl.when(i < NFULL)
        def _():
            pltpu.sync_copy(zdeg, acc.at[pl.ds(i * CH, CH)])

    @pl.when(s == NS - 1)
    def _():
        pltpu.sync_copy(zdeg.at[pl.ds(0, NTAIL)], acc.at[pl.ds(NFULL * CH, NTAIL)])

    for r in range(CH // 16):
        ones_v[pl.ds(r * 16, 16)] = jnp.ones((16,), jnp.float32)
    ones_t[...] = jnp.ones((8,), jnp.float32)

    for b in range(2):
        pltpu.async_copy(dst_hbm.at[pl.ds(ebase + b * CH, CH)], dv[b], hsem[b])

    nk = EMB_NCH // (NC * NS) + 1

    for k in range(nk + 1):
        if k < nk:
            ci = w + k * NC * NS
            b = k % 2

            @pl.when(ci < EMB_NCH)
            def _():
                pltpu.sync_copy(x_hbm.at[pl.ds(ci * EMB_CH, EMB_CH)], xv[b])
                pltpu.async_copy(emb_hbm.at[xv[b]], rows[b], esem[b])
        if k >= 1:
            pci = w + (k - 1) * NC * NS
            pb = (k - 1) % 2

            @pl.when(pci < EMB_NCH)
            def _():
                pltpu.make_async_copy(emb_hbm.at[xv[pb]], rows[pb],
                                      esem[pb]).wait()
                pltpu.sync_copy(rows[pb], h0_hbm.at[pl.ds(pci * EMB_CH, EMB_CH)])

    plsc.subcore_barrier()

    @pl.loop(0, HIST_NCH // 2)
    def _(i):
        for b in range(2):
            j = 2 * i + b
            pltpu.make_async_copy(dst_hbm.at[pl.ds(ebase, CH)], dv[b],
                                  hsem[b]).wait()
            pltpu.sync_copy(ones_v, acc.at[dv[b]], add=True)
            j2 = j + 2

            @pl.when(j2 < HIST_NCH)
            def _():
                pltpu.async_copy(dst_hbm.at[pl.ds(ebase + j2 * CH, CH)],
                                 dv[b], hsem[b])

    if HIST_NCH % 2 == 1:
        pltpu.make_async_copy(dst_hbm.at[pl.ds(ebase, CH)], dv[0], hsem[0]).wait()
        pltpu.sync_copy(ones_v, acc.at[dv[0]], add=True)

    rem = EPW - HIST_NCH * CH
    pltpu.sync_copy(dst_hbm.at[pl.ds(ebase + EPW - rem, rem)], dtail)
    pltpu.sync_copy(ones_t, acc.at[dtail], add=True)

    plsc.subcore_barrier()

    @pl.when(s == 0)
    def _():
        pltpu.sync_copy(acc, degp_hbm.at[c])


def _sc_prep(x, dst, emb):
    return pl.kernel(
        _sc_prep_body,
        out_type=(jax.ShapeDtypeStruct((N, D), jnp.float32),
                  jax.ShapeDtypeStruct((NC, N), jnp.float32)),
        mesh=_mesh(),
        scratch_types=[
            pltpu.VMEM((EMB_CH,), jnp.int32),
            pltpu.VMEM((EMB_CH,), jnp.int32),
            pltpu.VMEM((EMB_CH, D), jnp.float32),
            pltpu.VMEM((EMB_CH, D), jnp.float32),
            pltpu.VMEM((CH,), jnp.int32),
            pltpu.VMEM((CH,), jnp.int32),
            pltpu.VMEM((CH,), jnp.float32),
            pltpu.VMEM((8,), jnp.int32),
            pltpu.VMEM((8,), jnp.float32),
            pltpu.VMEM((CH,), jnp.float32),
            pltpu.VMEM_SHARED((N,), jnp.float32),
            pltpu.SemaphoreType.DMA,
            pltpu.SemaphoreType.DMA,
            pltpu.SemaphoreType.DMA,
            pltpu.SemaphoreType.DMA,
        ],
    )(x, dst, emb)


ROWS_PT = (E // CH) // NS
EPT_FULL = ROWS_PT * CH
EXTRA_OFF = NS * EPT_FULL
N_EXTRA = (E - NS * EPT_FULL) // CH
NB = 3


def _sc_agg_body(g_hbm, src_hbm, dst_hbm, zeros_hbm, s_hbm,
                 s0, s1, s2, v0, v1, v2, d0, d1, d2, r0, r1, r2, acc,
                 gs0, gs1, gs2, ds0, ds1, ds2, xs0, xs1, xs2):
    c = lax.axis_index("c")
    s = lax.axis_index("s")
    cN = c * N
    sv = (s0, s1, s2)
    svo = (v0, v1, v2)
    dv = (d0, d1, d2)
    rows = (r0, r1, r2)
    gsem = (gs0, gs1, gs2)
    dsem = (ds0, ds1, ds2)
    xsem = (xs0, xs1, xs2)

    def zero_chunks(do):
        for k in range(NFULL // NS + 1):
            i = s + k * NS

            @pl.when(i < NFULL)
            def _():
                do(zeros_hbm.at[pl.ds(i * CH, CH)], acc.at[pl.ds(i * CH, CH)])

        @pl.when(s == NS - 1)
        def _():
            do(zeros_hbm.at[pl.ds(NFULL * CH, NTAIL)],
               acc.at[pl.ds(NFULL * CH, NTAIL)])

    zero_chunks(lambda a, b: pltpu.async_copy(a, b, gs0))
    zero_chunks(lambda a, b: pltpu.make_async_copy(a, b, gs0).wait())

    ebase = s * EPT_FULL
    nch = jnp.where(s < N_EXTRA, ROWS_PT + 1, ROWS_PT)

    def doff(j):
        return jnp.where(j < ROWS_PT, ebase + j * CH, EXTRA_OFF + s * CH)

    def add_off(b):
        for q in range(CH // 16):
            svo[b][pl.ds(q * 16, 16)] = sv[b][pl.ds(q * 16, 16)] + cN

    plsc.subcore_barrier()

    for b in range(NB):
        pltpu.async_copy(dst_hbm.at[pl.ds(doff(b), CH)], dv[b], dsem[b])
        pltpu.async_copy(src_hbm.at[pl.ds(doff(b), CH)], sv[b], xsem[b])
    for b in range(NB):
        pltpu.make_async_copy(src_hbm.at[pl.ds(doff(b), CH)], sv[b],
                              xsem[b]).wait()
        add_off(b)
        pltpu.async_copy(g_hbm.at[svo[b]], rows[b], gsem[b])
    for b in range(NB):
        pltpu.async_copy(src_hbm.at[pl.ds(doff(b + NB), CH)], sv[b], xsem[b])

    @pl.loop(0, ROWS_PT // NB)
    def _(i):
        for b in range(NB):
            j = i * NB + b
            jn = j + NB
            pltpu.make_async_copy(dst_hbm.at[pl.ds(doff(j), CH)], dv[b],
                                  dsem[b]).wait()
            pltpu.make_async_copy(g_hbm.at[svo[b]], rows[b], gsem[b]).wait()
            pltpu.sync_copy(rows[b], acc.at[dv[b]], add=True)

            @pl.when(jn < nch)
            def _():
                pltpu.make_async_copy(src_hbm.at[pl.ds(doff(jn), CH)], sv[b],
                                      xsem[b]).wait()
                add_off(b)
                pltpu.async_copy(g_hbm.at[svo[b]], rows[b], gsem[b])
                pltpu.async_copy(dst_hbm.at[pl.ds(doff(jn), CH)], dv[b], dsem[b])
                jnn = jn + NB

                @pl.when(jnn < nch)
                def _():
                    pltpu.async_copy(src_hbm.at[pl.ds(doff(jnn), CH)], sv[b],
                                     xsem[b])

    for b in range(ROWS_PT % NB + 1):
        j = (ROWS_PT // NB) * NB + b

        @pl.when(j < nch)
        def _():
            pltpu.make_async_copy(dst_hbm.at[pl.ds(doff(j), CH)], dv[b],
                                  dsem[b]).wait()
            pltpu.make_async_copy(g_hbm.at[svo[b]], rows[b], gsem[b]).wait()
            pltpu.sync_copy(rows[b], acc.at[dv[b]], add=True)

    plsc.subcore_barrier()

    def out_chunks(do):
        for k in range(NFULL // NS + 1):
            i = s + k * NS

            @pl.when(i < NFULL)
            def _():
                do(acc.at[pl.ds(i * CH, CH)], s_hbm.at[c, pl.ds(i * CH, CH)])

        @pl.when(s == NS - 1)
        def _():
            do(acc.at[pl.ds(NFULL * CH, NTAIL)],
               s_hbm.at[c, pl.ds(NFULL * CH, NTAIL)])

    out_chunks(lambda a, b: pltpu.async_copy(a, b, gs0))
    out_chunks(lambda a, b: pltpu.make_async_copy(a, b, gs0).wait())


def _sc_agg(g2, src, dst, zeros):
    return pl.kernel(
        _sc_agg_body,
        out_type=jax.ShapeDtypeStruct((NC, N, H), jnp.float32),
        mesh=_mesh(),
        scratch_types=(
            [pltpu.VMEM((CH,), jnp.int32) for _ in range(NB)]
            + [pltpu.VMEM((CH,), jnp.int32) for _ in range(NB)]
            + [pltpu.VMEM((CH,), jnp.int32) for _ in range(NB)]
            + [pltpu.VMEM((CH, H), jnp.float32) for _ in range(NB)]
            + [pltpu.VMEM_SHARED((N, H), jnp.float32)]
            + [pltpu.SemaphoreType.DMA for _ in range(3 * NB)]
        ),
    )(g2, src, dst, zeros)


BLK = 2000


def _tc0_body(degp_ref, h0_ref, w_ref, g_ref, dinv_ref):
    deg = degp_ref[0] + degp_ref[1] + 1.0
    dinv = lax.rsqrt(deg)
    dinv_ref[...] = dinv
    u = h0_ref[...] * dinv
    g = jnp.dot(u, w_ref[...], preferred_element_type=jnp.float32)
    g_ref[0] = g[:, :H]
    g_ref[1] = g[:, H:]


def _tc0(degp, h0, W1):
    return pl.pallas_call(
        _tc0_body,
        grid=(N // BLK,),
        in_specs=[
            pl.BlockSpec((NC, BLK, 1), lambda i: (0, i, 0)),
            pl.BlockSpec((BLK, D), lambda i: (i, 0)),
            pl.BlockSpec((D, D), lambda i: (0, 0)),
        ],
        out_specs=[
            pl.BlockSpec((NC, BLK, H), lambda i: (0, i, 0)),
            pl.BlockSpec((BLK, 1), lambda i: (i, 0)),
        ],
        out_shape=[
            jax.ShapeDtypeStruct((NC, N, H), jnp.float32),
            jax.ShapeDtypeStruct((N, 1), jnp.float32),
        ],
    )(degp, h0, W1)


def _tc_mid_body(s_ref, g_ref, dinv_ref, b_ref, w_ref, o_ref):
    dinv = dinv_ref[...]
    t = jnp.concatenate([s_ref[0] + g_ref[0], s_ref[1] + g_ref[1]], axis=1)
    h = jnp.maximum(t * dinv + b_ref[...], 0.0)
    g = jnp.dot(h * dinv, w_ref[...], preferred_element_type=jnp.float32)
    o_ref[0] = g[:, :H]
    o_ref[1] = g[:, H:]


def _tc_mid(s, g, dinv, b, W):
    return pl.pallas_call(
        _tc_mid_body,
        grid=(N // BLK,),
        in_specs=[
            pl.BlockSpec((NC, BLK, H), lambda i: (0, i, 0)),
            pl.BlockSpec((NC, BLK, H), lambda i: (0, i, 0)),
            pl.BlockSpec((BLK, 1), lambda i: (i, 0)),
            pl.BlockSpec((1, D), lambda i: (0, 0)),
            pl.BlockSpec((D, D), lambda i: (0, 0)),
        ],
        out_specs=pl.BlockSpec((NC, BLK, H), lambda i: (0, i, 0)),
        out_shape=jax.ShapeDtypeStruct((NC, N, H), jnp.float32),
    )(s, g, dinv, b, W)


def _tc_fin_body(s_ref, g_ref, dinv_ref, b_ref, o_ref):
    t = jnp.concatenate([s_ref[0] + g_ref[0], s_ref[1] + g_ref[1]], axis=1)
    o_ref[...] = t * dinv_ref[...] + b_ref[...]


def _tc_fin(s, g, dinv, b):
    return pl.pallas_call(
        _tc_fin_body,
        grid=(N // BLK,),
        in_specs=[
            pl.BlockSpec((NC, BLK, H), lambda i: (0, i, 0)),
            pl.BlockSpec((NC, BLK, H), lambda i: (0, i, 0)),
            pl.BlockSpec((BLK, 1), lambda i: (i, 0)),
            pl.BlockSpec((1, D), lambda i: (0, 0)),
        ],
        out_specs=pl.BlockSpec((BLK, D), lambda i: (i, 0)),
        out_shape=jax.ShapeDtypeStruct((N, D), jnp.float32),
    )(s, g, dinv, b)


def kernel(x, edge_index, emb_table, W1, b1, W2, b2, W3, b3):
    x = x.astype(jnp.int32)
    src = edge_index[0].astype(jnp.int32)
    dst = edge_index[1].astype(jnp.int32)
    zeros = jnp.zeros((N, H), jnp.float32)

    h0, degp = _sc_prep(x, dst, emb_table)
    degp3 = degp.reshape(NC, N, 1)

    g1, dinv = _tc0(degp3, h0, W1)
    s1 = _sc_agg(g1.reshape(NC * N, H), src, dst, zeros)

    g2 = _tc_mid(s1, g1, dinv, b1.reshape(1, D), W2)
    s2 = _sc_agg(g2.reshape(NC * N, H), src, dst, zeros)

    g3 = _tc_mid(s2, g2, dinv, b2.reshape(1, D), W3)
    s3 = _sc_agg(g3.reshape(NC * N, H), src, dst, zeros)

    return _tc_fin(s3, g3, dinv, b3.reshape(1, D))


# speedup vs baseline: 1.0311x; 1.0064x over previous
import jax
import jax.numpy as jnp
from jax import lax
from jax.experimental import pallas as pl
from jax.experimental.pallas import tpu as pltpu
from jax.experimental.pallas import tpu_sc as plsc

N = 10000
E = 160000
D = 256
H = 128
NC = 2
NS = 16
CH = 128

EPW = E // (NC * NS)
EPT = E // NS
NFULL = N // CH
NTAIL = N - NFULL * CH

def _mesh():
    return plsc.VectorSubcoreMesh(core_axis_name="c", subcore_axis_name="s",
                                  num_cores=NC, num_subcores=NS)


def _zero_vmem(ref, nelem):
    z = jnp.zeros((16,), jnp.float32)

    @pl.loop(0, nelem // 16)
    def _(i):
        ref[pl.ds(i * 16, 16)] = z


EMB_CH = 80
EMB_NCH = N // EMB_CH
HIST_NCH = EPW // CH


def _sc_prep_body(x_hbm, dst_hbm, emb_hbm, h0_hbm, degp_hbm,
                  xv0, xv1, rows0, rows1, dv0, dv1, ones_v, dtail, ones_t,
                  zdeg, acc, es0, es1, hs0, hs1):
    c = lax.axis_index("c")
    s = lax.axis_index("s")
    w = s * NC + c
    xv = (xv0, xv1)
    rows = (rows0, rows1)
    dv = (dv0, dv1)
    esem = (es0, es1)
    hsem = (hs0, hs1)
    ebase = w * EPW

    _zero_vmem(zdeg, CH)
    for k in range(NFULL // NS + 1):
        i = s + k * NS

        @pl.when(i < NFULL)
        def _():
            pltpu.sync_copy(zdeg, acc.at[pl.ds(i * CH, CH)])

    @pl.when(s == NS - 1)
    def _():
        pltpu.sync_copy(zdeg.at[pl.ds(0, NTAIL)], acc.at[pl.ds(NFULL * CH, NTAIL)])

    for r in range(CH // 16):
        ones_v[pl.ds(r * 16, 16)] = jnp.ones((16,), jnp.float32)
    ones_t[...] = jnp.ones((8,), jnp.float32)

    for b in range(2):
        pltpu.async_copy(dst_hbm.at[pl.ds(ebase + b * CH, CH)], dv[b], hsem[b])

    nk = EMB_NCH // (NC * NS) + 1

    for k in range(nk + 1):
        if k < nk:
            ci = w + k * NC * NS
            b = k % 2

            @pl.when(ci < EMB_NCH)
            def _():
                pltpu.sync_copy(x_hbm.at[pl.ds(ci * EMB_CH, EMB_CH)], xv[b])
                pltpu.async_copy(emb_hbm.at[xv[b]], rows[b], esem[b])
        if k >= 1:
            pci = w + (k - 1) * NC * NS
            pb = (k - 1) % 2

            @pl.when(pci < EMB_NCH)
            def _():
                pltpu.make_async_copy(emb_hbm.at[xv[pb]], rows[pb],
                                      esem[pb]).wait()
                pltpu.sync_copy(rows[pb], h0_hbm.at[pl.ds(pci * EMB_CH, EMB_CH)])

    plsc.subcore_barrier()

    @pl.loop(0, HIST_NCH // 2)
    def _(i):
        for b in range(2):
            j = 2 * i + b
            pltpu.make_async_copy(dst_hbm.at[pl.ds(ebase, CH)], dv[b],
                                  hsem[b]).wait()
            pltpu.sync_copy(ones_v, acc.at[dv[b]], add=True)
            j2 = j + 2

            @pl.when(j2 < HIST_NCH)
            def _():
                pltpu.async_copy(dst_hbm.at[pl.ds(ebase + j2 * CH, CH)],
                                 dv[b], hsem[b])

    if HIST_NCH % 2 == 1:
        pltpu.make_async_copy(dst_hbm.at[pl.ds(ebase, CH)], dv[0], hsem[0]).wait()
        pltpu.sync_copy(ones_v, acc.at[dv[0]], add=True)

    rem = EPW - HIST_NCH * CH
    pltpu.sync_copy(dst_hbm.at[pl.ds(ebase + EPW - rem, rem)], dtail)
    pltpu.sync_copy(ones_t, acc.at[dtail], add=True)

    plsc.subcore_barrier()

    @pl.when(s == 0)
    def _():
        pltpu.sync_copy(acc, degp_hbm.at[c])


def _sc_prep(x, dst, emb):
    return pl.kernel(
        _sc_prep_body,
        out_type=(jax.ShapeDtypeStruct((N, D), jnp.float32),
                  jax.ShapeDtypeStruct((NC, N), jnp.float32)),
        mesh=_mesh(),
        scratch_types=[
            pltpu.VMEM((EMB_CH,), jnp.int32),
            pltpu.VMEM((EMB_CH,), jnp.int32),
            pltpu.VMEM((EMB_CH, D), jnp.float32),
            pltpu.VMEM((EMB_CH, D), jnp.float32),
            pltpu.VMEM((CH,), jnp.int32),
            pltpu.VMEM((CH,), jnp.int32),
            pltpu.VMEM((CH,), jnp.float32),
            pltpu.VMEM((8,), jnp.int32),
            pltpu.VMEM((8,), jnp.float32),
            pltpu.VMEM((CH,), jnp.float32),
            pltpu.VMEM_SHARED((N,), jnp.float32),
            pltpu.SemaphoreType.DMA,
            pltpu.SemaphoreType.DMA,
            pltpu.SemaphoreType.DMA,
            pltpu.SemaphoreType.DMA,
        ],
    )(x, dst, emb)


ROWS_PT = (E // CH) // NS
EPT_FULL = ROWS_PT * CH
EXTRA_OFF = NS * EPT_FULL
N_EXTRA = (E - NS * EPT_FULL) // CH
NB = 3


def _sc_agg_body(g_hbm, src_hbm, dst_hbm, zeros_hbm, s_hbm,
                 s0, s1, s2, v0, v1, v2, d0, d1, d2, r0, r1, r2, acc,
                 gs0, gs1, gs2, ds0, ds1, ds2, xs0, xs1, xs2):
    c = lax.axis_index("c")
    s = lax.axis_index("s")
    cN = c * N
    sv = (s0, s1, s2)
    svo = (v0, v1, v2)
    dv = (d0, d1, d2)
    rows = (r0, r1, r2)
    gsem = (gs0, gs1, gs2)
    dsem = (ds0, ds1, ds2)
    xsem = (xs0, xs1, xs2)

    def zero_chunks(do):
        for k in range(NFULL // NS + 1):
            i = s + k * NS

            @pl.when(i < NFULL)
            def _():
                do(zeros_hbm.at[pl.ds(i * CH, CH)], acc.at[pl.ds(i * CH, CH)])

        @pl.when(s == NS - 1)
        def _():
            do(zeros_hbm.at[pl.ds(NFULL * CH, NTAIL)],
               acc.at[pl.ds(NFULL * CH, NTAIL)])

    zero_chunks(lambda a, b: pltpu.async_copy(a, b, gs0))
    zero_chunks(lambda a, b: pltpu.make_async_copy(a, b, gs0).wait())

    ebase = s * EPT_FULL
    nch = jnp.where(s < N_EXTRA, ROWS_PT + 1, ROWS_PT)

    def doff(j):
        return jnp.where(j < ROWS_PT, ebase + j * CH, EXTRA_OFF + s * CH)

    def add_off(b):
        for q in range(CH // 16):
            svo[b][pl.ds(q * 16, 16)] = sv[b][pl.ds(q * 16, 16)] + cN

    plsc.subcore_barrier()

    for b in range(NB):
        pltpu.async_copy(dst_hbm.at[pl.ds(doff(b), CH)], dv[b], dsem[b])
        pltpu.async_copy(src_hbm.at[pl.ds(doff(b), CH)], sv[b], xsem[b])
    for b in range(NB):
        pltpu.make_async_copy(src_hbm.at[pl.ds(doff(b), CH)], sv[b],
                              xsem[b]).wait()
        add_off(b)
        pltpu.async_copy(g_hbm.at[svo[b]], rows[b], gsem[b])
    for b in range(NB):
        pltpu.async_copy(src_hbm.at[pl.ds(doff(b + NB), CH)], sv[b], xsem[b])

    @pl.loop(0, ROWS_PT // NB)
    def _(i):
        for b in range(NB):
            j = i * NB + b
            jn = j + NB
            pltpu.make_async_copy(dst_hbm.at[pl.ds(doff(j), CH)], dv[b],
                                  dsem[b]).wait()
            pltpu.make_async_copy(g_hbm.at[svo[b]], rows[b], gsem[b]).wait()
            pltpu.sync_copy(rows[b], acc.at[dv[b]], add=True)

            @pl.when(jn < nch)
            def _():
                pltpu.make_async_copy(src_hbm.at[pl.ds(doff(jn), CH)], sv[b],
                                      xsem[b]).wait()
                add_off(b)
                pltpu.async_copy(g_hbm.at[svo[b]], rows[b], gsem[b])
                pltpu.async_copy(dst_hbm.at[pl.ds(doff(jn), CH)], dv[b], dsem[b])
                jnn = jn + NB

                @pl.when(jnn < nch)
                def _():
                    pltpu.async_copy(src_hbm.at[pl.ds(doff(jnn), CH)], sv[b],
                                     xsem[b])

    for b in range(ROWS_PT % NB + 1):
        j = (ROWS_PT // NB) * NB + b

        @pl.when(j < nch)
        def _():
            pltpu.make_async_copy(dst_hbm.at[pl.ds(doff(j), CH)], dv[b],
                                  dsem[b]).wait()
            pltpu.make_async_copy(g_hbm.at[svo[b]], rows[b], gsem[b]).wait()
            pltpu.sync_copy(rows[b], acc.at[dv[b]], add=True)

    plsc.subcore_barrier()

    def out_chunks(do):
        for k in range(NFULL // NS + 1):
            i = s + k * NS

            @pl.when(i < NFULL)
            def _():
                do(acc.at[pl.ds(i * CH, CH)], s_hbm.at[c, pl.ds(i * CH, CH)])

        @pl.when(s == NS - 1)
        def _():
            do(acc.at[pl.ds(NFULL * CH, NTAIL)],
               s_hbm.at[c, pl.ds(NFULL * CH, NTAIL)])

    out_chunks(lambda a, b: pltpu.async_copy(a, b, gs0))
    out_chunks(lambda a, b: pltpu.make_async_copy(a, b, gs0).wait())


def _sc_agg(g2, src, dst, zeros):
    return pl.kernel(
        _sc_agg_body,
        out_type=jax.ShapeDtypeStruct((NC, N, H), jnp.float32),
        mesh=_mesh(),
        scratch_types=(
            [pltpu.VMEM((CH,), jnp.int32) for _ in range(NB)]
            + [pltpu.VMEM((CH,), jnp.int32) for _ in range(NB)]
            + [pltpu.VMEM((CH,), jnp.int32) for _ in range(NB)]
            + [pltpu.VMEM((CH, H), jnp.float32) for _ in range(NB)]
            + [pltpu.VMEM_SHARED((N, H), jnp.float32)]
            + [pltpu.SemaphoreType.DMA for _ in range(3 * NB)]
        ),
    )(g2, src, dst, zeros)


BLK = 5000


def _tc0_body(degp_ref, h0_ref, w_ref, g_ref, dinv_ref):
    deg = degp_ref[0] + degp_ref[1] + 1.0
    dinv = lax.rsqrt(deg)
    dinv_ref[...] = dinv
    u = h0_ref[...] * dinv
    g = jnp.dot(u, w_ref[...], preferred_element_type=jnp.float32)
    g_ref[0] = g[:, :H]
    g_ref[1] = g[:, H:]


def _tc0(degp, h0, W1):
    return pl.pallas_call(
        _tc0_body,
        grid=(N // BLK,),
        in_specs=[
            pl.BlockSpec((NC, BLK, 1), lambda i: (0, i, 0)),
            pl.BlockSpec((BLK, D), lambda i: (i, 0)),
            pl.BlockSpec((D, D), lambda i: (0, 0)),
        ],
        out_specs=[
            pl.BlockSpec((NC, BLK, H), lambda i: (0, i, 0)),
            pl.BlockSpec((BLK, 1), lambda i: (i, 0)),
        ],
        out_shape=[
            jax.ShapeDtypeStruct((NC, N, H), jnp.float32),
            jax.ShapeDtypeStruct((N, 1), jnp.float32),
        ],
    )(degp, h0, W1)


def _tc_mid_body(s_ref, g_ref, dinv_ref, b_ref, w_ref, o_ref):
    dinv = dinv_ref[...]
    t = jnp.concatenate([s_ref[0] + g_ref[0], s_ref[1] + g_ref[1]], axis=1)
    h = jnp.maximum(t * dinv + b_ref[...], 0.0)
    g = jnp.dot(h * dinv, w_ref[...], preferred_element_type=jnp.float32)
    o_ref[0] = g[:, :H]
    o_ref[1] = g[:, H:]


def _tc_mid(s, g, dinv, b, W):
    return pl.pallas_call(
        _tc_mid_body,
        grid=(N // BLK,),
        in_specs=[
            pl.BlockSpec((NC, BLK, H), lambda i: (0, i, 0)),
            pl.BlockSpec((NC, BLK, H), lambda i: (0, i, 0)),
            pl.BlockSpec((BLK, 1), lambda i: (i, 0)),
            pl.BlockSpec((1, D), lambda i: (0, 0)),
            pl.BlockSpec((D, D), lambda i: (0, 0)),
        ],
        out_specs=pl.BlockSpec((NC, BLK, H), lambda i: (0, i, 0)),
        out_shape=jax.ShapeDtypeStruct((NC, N, H), jnp.float32),
    )(s, g, dinv, b, W)


def _tc_fin_body(s_ref, g_ref, dinv_ref, b_ref, o_ref):
    t = jnp.concatenate([s_ref[0] + g_ref[0], s_ref[1] + g_ref[1]], axis=1)
    o_ref[...] = t * dinv_ref[...] + b_ref[...]


def _tc_fin(s, g, dinv, b):
    return pl.pallas_call(
        _tc_fin_body,
        grid=(N // BLK,),
        in_specs=[
            pl.BlockSpec((NC, BLK, H), lambda i: (0, i, 0)),
            pl.BlockSpec((NC, BLK, H), lambda i: (0, i, 0)),
            pl.BlockSpec((BLK, 1), lambda i: (i, 0)),
            pl.BlockSpec((1, D), lambda i: (0, 0)),
        ],
        out_specs=pl.BlockSpec((BLK, D), lambda i: (i, 0)),
        out_shape=jax.ShapeDtypeStruct((N, D), jnp.float32),
    )(s, g, dinv, b)


def kernel(x, edge_index, emb_table, W1, b1, W2, b2, W3, b3):
    x = x.astype(jnp.int32)
    src = edge_index[0].astype(jnp.int32)
    dst = edge_index[1].astype(jnp.int32)
    zeros = jnp.zeros((N, H), jnp.float32)

    h0, degp = _sc_prep(x, dst, emb_table)
    degp3 = degp.reshape(NC, N, 1)

    g1, dinv = _tc0(degp3, h0, W1)
    s1 = _sc_agg(g1.reshape(NC * N, H), src, dst, zeros)

    g2 = _tc_mid(s1, g1, dinv, b1.reshape(1, D), W2)
    s2 = _sc_agg(g2.reshape(NC * N, H), src, dst, zeros)

    g3 = _tc_mid(s2, g2, dinv, b2.reshape(1, D), W3)
    s3 = _sc_agg(g3.reshape(NC * N, H), src, dst, zeros)

    return _tc_fin(s3, g3, dinv, b3.reshape(1, D))
